# baseline (speedup 1.0000x reference)
import functools

import jax
import jax.numpy as jnp
from jax import lax
from jax.experimental import pallas as pl
from jax.experimental.pallas import tpu as pltpu
from jax.experimental.pallas import tpu_sc as plsc

_L = 16
_NW = 32


def _comb_body(emb_ref, sp_ref, out_ref):
    out_ref[...] = sp_ref[...][:, None, :] + emb_ref[...][None, :, :]


def _make_comb(embedding, species_embedding):
    S, H = species_embedding.shape
    comb = pl.pallas_call(
        _comb_body,
        out_shape=jax.ShapeDtypeStruct((S, 3, H), jnp.float32),
    )(embedding, species_embedding)
    return comb.reshape(S * 3, H)


def _tc_tail_body(ids_ref, emb_ref, sp_ref, out_ref):
    ids = ids_ref[...][..., None]
    e0 = emb_ref[0]
    e1 = emb_ref[1]
    e2 = emb_ref[2]
    sp = sp_ref[...][None, :, None, :]
    out_ref[...] = jnp.where(ids == 0, e0, jnp.where(ids == 1, e1, e2)) + sp


def _tc_tail(ids_tail, embedding, species_embedding):
    B, S, TT = ids_tail.shape
    H = embedding.shape[1]
    return pl.pallas_call(
        _tc_tail_body,
        grid=(B,),
        in_specs=[
            pl.BlockSpec((1, S, TT), lambda b: (b, 0, 0)),
            pl.BlockSpec((3, H), lambda b: (0, 0)),
            pl.BlockSpec((S, H), lambda b: (0, 0)),
        ],
        out_specs=pl.BlockSpec((1, S, TT, H), lambda b: (b, 0, 0, 0)),
        out_shape=jax.ShapeDtypeStruct((B, S, TT, H), jnp.float32),
    )(ids_tail, embedding, species_embedding)


def _make_sc_kernel(B, S, T, H):
    nslab = B * S
    per_w = nslab // _NW
    t0 = (T // 8) * 8
    mesh = plsc.VectorSubcoreMesh(core_axis_name="c", subcore_axis_name="s")

    @functools.partial(
        pl.kernel,
        out_type=jax.ShapeDtypeStruct((B, S, T, H), jnp.float32),
        mesh=mesh,
        scratch_types=[
            pltpu.VMEM((per_w * T,), jnp.int32),
            pltpu.VMEM((2, t0), jnp.int32),
            pltpu.VMEM((2, t0, H), jnp.float32),
            pltpu.SemaphoreType.DMA,
            pltpu.SemaphoreType.DMA,
            pltpu.SemaphoreType.DMA,
        ],
        compiler_params=pltpu.CompilerParams(
            use_tc_tiling_on_sc=True, needs_layout_passes=False
        ),
    )
    def sc_k(ids_hbm, comb_hbm, out_hbm, idsv, keys, rows, sem_g, sem_s0,
             sem_s1):
        wid = lax.axis_index("s") * 2 + lax.axis_index("c")
        nids = per_w * T
        pltpu.sync_copy(ids_hbm.at[pl.ds(wid * nids, nids)], idsv)
        sems = [sem_s0, sem_s1]

        def slab(j, b, sem_b, first):
            g = wid * per_w + j
            bi = lax.div(g, jnp.int32(S))
            si = lax.rem(g, jnp.int32(S))
            for i in range(t0 // _L):
                keys[b, pl.ds(i * _L, _L)] = (
                    idsv[pl.ds(j * T + i * _L, _L)] + si * 3
                )

            @pl.when(jnp.logical_not(first))
            def _():
                pltpu.make_async_copy(
                    rows.at[b], out_hbm.at[0, 0, pl.ds(0, t0)], sem_b
                ).wait()

            pltpu.async_copy(comb_hbm.at[keys.at[b]], rows.at[b], sem_g).wait()
            pltpu.async_copy(
                rows.at[b], out_hbm.at[bi, si, pl.ds(0, t0)], sem_b
            )

        def body(jj, carry):
            for b in range(2):
                slab(jj * 2 + b, b, sems[b], jj == 0)
            return carry

        lax.fori_loop(0, per_w // 2, body, 0)
        for b in range(2):
            pltpu.make_async_copy(
                rows.at[b], out_hbm.at[0, 0, pl.ds(0, t0)], sems[b]
            ).wait()

    return sc_k


def kernel(input_ids, embedding, species_embedding):
    B, S, T = input_ids.shape
    H = embedding.shape[1]
    t0 = (T // 8) * 8
    comb = _make_comb(embedding, species_embedding)
    ids_flat = input_ids.reshape(B * S * T)
    sc_k = _make_sc_kernel(B, S, T, H)
    big = sc_k(ids_flat, comb)
    tail = _tc_tail(input_ids[:, :, t0:], embedding, species_embedding)
    return lax.dynamic_update_slice(big, tail, (0, 0, t0, 0))


# speedup vs baseline: 1.1327x; 1.1327x over previous
import functools

import jax
import jax.numpy as jnp
from jax import lax
from jax.experimental import pallas as pl
from jax.experimental.pallas import tpu as pltpu
from jax.experimental.pallas import tpu_sc as plsc

_L = 16
_NW = 32


def _comb_body(emb_ref, sp_ref, out_ref):
    out_ref[...] = sp_ref[...][:, None, :] + emb_ref[...][None, :, :]


def _make_comb(embedding, species_embedding):
    S, H = species_embedding.shape
    comb = pl.pallas_call(
        _comb_body,
        out_shape=jax.ShapeDtypeStruct((S, 3, H), jnp.float32),
    )(embedding, species_embedding)
    return comb.reshape(S * 3, H)


def _tc_tail_body(ids_ref, emb_ref, sp_ref, out_ref):
    ids = ids_ref[...][..., None]
    e0 = emb_ref[0]
    e1 = emb_ref[1]
    e2 = emb_ref[2]
    sp = sp_ref[...][None, :, None, :]
    out_ref[...] = jnp.where(ids == 0, e0, jnp.where(ids == 1, e1, e2)) + sp


def _tc_tail(ids_tail, embedding, species_embedding):
    B, S, TT = ids_tail.shape
    H = embedding.shape[1]
    return pl.pallas_call(
        _tc_tail_body,
        grid=(B,),
        in_specs=[
            pl.BlockSpec((1, S, TT), lambda b: (b, 0, 0)),
            pl.BlockSpec((3, H), lambda b: (0, 0)),
            pl.BlockSpec((S, H), lambda b: (0, 0)),
        ],
        out_specs=pl.BlockSpec((1, S, TT, H), lambda b: (b, 0, 0, 0)),
        out_shape=jax.ShapeDtypeStruct((B, S, TT, H), jnp.float32),
    )(ids_tail, embedding, species_embedding)


def _make_sc_kernel(B, S, T, H):
    nslab = B * S
    per_w = nslab // _NW
    t0 = (T // 8) * 8
    mesh = plsc.VectorSubcoreMesh(core_axis_name="c", subcore_axis_name="s")

    @functools.partial(
        pl.kernel,
        out_type=jax.ShapeDtypeStruct((B, S, T, H), jnp.float32),
        mesh=mesh,
        scratch_types=[
            pltpu.VMEM((per_w * T,), jnp.int32),
            pltpu.VMEM((4, t0), jnp.int32),
            pltpu.VMEM((4, t0, H), jnp.float32),
            [pltpu.SemaphoreType.DMA] * 4,
            [pltpu.SemaphoreType.DMA] * 4,
        ],
        compiler_params=pltpu.CompilerParams(
            use_tc_tiling_on_sc=True, needs_layout_passes=False
        ),
    )
    def sc_k(ids_hbm, comb_hbm, out_hbm, idsv, keys, rows, sgs, sss):
        wid = lax.axis_index("s") * 2 + lax.axis_index("c")
        nids = per_w * T
        pltpu.sync_copy(ids_hbm.at[pl.ds(wid * nids, nids)], idsv)

        def fire(jn, s, guard_scatter):
            g = wid * per_w + jn
            si = lax.rem(g, jnp.int32(S))
            for i in range(t0 // _L):
                keys[s, pl.ds(i * _L, _L)] = (
                    idsv[pl.ds(jn * T + i * _L, _L)] + si * 3
                )
            if guard_scatter:
                @pl.when(jn >= 4)
                def _():
                    pltpu.make_async_copy(
                        rows.at[s], out_hbm.at[0, 0, pl.ds(0, t0)], sss[s]
                    ).wait()

            pltpu.async_copy(comb_hbm.at[keys.at[s]], rows.at[s], sgs[s])

        def drain(j, s):
            g = wid * per_w + j
            bi = lax.div(g, jnp.int32(S))
            si = lax.rem(g, jnp.int32(S))
            pltpu.make_async_copy(
                comb_hbm.at[keys.at[s]], rows.at[s], sgs[s]
            ).wait()
            pltpu.async_copy(rows.at[s], out_hbm.at[bi, si, pl.ds(0, t0)],
                             sss[s])

        fire(0, 0, False)
        fire(1, 1, False)

        def body(jj, carry):
            for b in range(4):
                j = jj * 4 + b
                jn = j + 2
                sn = (b + 2) % 4

                @pl.when(jn < per_w)
                def _():
                    fire(jn, sn, True)

                drain(j, b)
            return carry

        lax.fori_loop(0, per_w // 4, body, 0)
        for s in range(4):
            pltpu.make_async_copy(
                rows.at[s], out_hbm.at[0, 0, pl.ds(0, t0)], sss[s]
            ).wait()

    return sc_k


def kernel(input_ids, embedding, species_embedding):
    B, S, T = input_ids.shape
    H = embedding.shape[1]
    t0 = (T // 8) * 8
    comb = _make_comb(embedding, species_embedding)
    ids_flat = input_ids.reshape(B * S * T)
    sc_k = _make_sc_kernel(B, S, T, H)
    big = sc_k(ids_flat, comb)
    tail = _tc_tail(input_ids[:, :, t0:], embedding, species_embedding)
    return lax.dynamic_update_slice(big, tail, (0, 0, t0, 0))


# speedup vs baseline: 1.1884x; 1.0492x over previous
import functools

import jax
import jax.numpy as jnp
from jax import lax
from jax.experimental import pallas as pl
from jax.experimental.pallas import tpu as pltpu
from jax.experimental.pallas import tpu_sc as plsc

_L = 16
_NW = 32


def _comb_body(emb_ref, sp_ref, out_ref):
    out_ref[...] = sp_ref[...][:, None, :] + emb_ref[...][None, :, :]


def _make_comb(embedding, species_embedding):
    S, H = species_embedding.shape
    comb = pl.pallas_call(
        _comb_body,
        out_shape=jax.ShapeDtypeStruct((S, 3, H), jnp.float32),
    )(embedding, species_embedding)
    return comb.reshape(S * 3, H)


def _tc_tail_body(ids_ref, emb_ref, sp_ref, out_ref):
    ids = ids_ref[...][..., None]
    e0 = emb_ref[0]
    e1 = emb_ref[1]
    e2 = emb_ref[2]
    sp = sp_ref[...][None, :, None, :]
    out_ref[...] = jnp.where(ids == 0, e0, jnp.where(ids == 1, e1, e2)) + sp


def _tc_tail(ids_tail, embedding, species_embedding):
    B, S, TT = ids_tail.shape
    H = embedding.shape[1]
    return pl.pallas_call(
        _tc_tail_body,
        grid=(B,),
        in_specs=[
            pl.BlockSpec((1, S, TT), lambda b: (b, 0, 0)),
            pl.BlockSpec((3, H), lambda b: (0, 0)),
            pl.BlockSpec((S, H), lambda b: (0, 0)),
        ],
        out_specs=pl.BlockSpec((1, S, TT, H), lambda b: (b, 0, 0, 0)),
        out_shape=jax.ShapeDtypeStruct((B, S, TT, H), jnp.float32),
    )(ids_tail, embedding, species_embedding)


def _make_sc_kernel(B, S, T, H):
    nslab = B * S
    per_w = nslab // _NW
    t0 = (T // 8) * 8
    mesh = plsc.VectorSubcoreMesh(core_axis_name="c", subcore_axis_name="s")

    @functools.partial(
        pl.kernel,
        out_type=jax.ShapeDtypeStruct((B, S, T, H), jnp.float32),
        mesh=mesh,
        scratch_types=[
            pltpu.VMEM((per_w * T,), jnp.int32),
            pltpu.VMEM((4, 2 * t0), jnp.int32),
            pltpu.VMEM((4, 2 * t0, H), jnp.float32),
            [pltpu.SemaphoreType.DMA] * 4,
            [pltpu.SemaphoreType.DMA] * 4,
        ],
        compiler_params=pltpu.CompilerParams(
            use_tc_tiling_on_sc=True, needs_layout_passes=False
        ),
    )
    def sc_k(ids_hbm, comb_hbm, out_hbm, idsv, keys, rows, sgs, sss):
        wid = lax.axis_index("s") * 2 + lax.axis_index("c")
        nids = per_w * T
        pltpu.sync_copy(ids_hbm.at[pl.ds(wid * nids, nids)], idsv)

        def fire(jn, s, guard_scatter):
            g = wid * per_w + 2 * jn
            for half in range(2):
                si = lax.rem(g + half, jnp.int32(S))
                for i in range(t0 // _L):
                    keys[s, pl.ds(half * t0 + i * _L, _L)] = (
                        idsv[pl.ds((2 * jn + half) * T + i * _L, _L)] + si * 3
                    )
            if guard_scatter:
                @pl.when(jn >= 4)
                def _():
                    for _k in range(2):
                        pltpu.make_async_copy(
                            rows.at[s, pl.ds(0, t0)],
                            out_hbm.at[0, 0, pl.ds(0, t0)],
                            sss[s],
                        ).wait()

            pltpu.async_copy(comb_hbm.at[keys.at[s]], rows.at[s], sgs[s])

        def drain(j, s):
            g = wid * per_w + 2 * j
            pltpu.make_async_copy(
                comb_hbm.at[keys.at[s]], rows.at[s], sgs[s]
            ).wait()
            for half in range(2):
                bi = lax.div(g + half, jnp.int32(S))
                si = lax.rem(g + half, jnp.int32(S))
                pltpu.async_copy(
                    rows.at[s, pl.ds(half * t0, t0)],
                    out_hbm.at[bi, si, pl.ds(0, t0)],
                    sss[s],
                )

        fire(0, 0, False)
        fire(1, 1, False)

        npair = per_w // 2

        def body(jj, carry):
            for b in range(4):
                j = jj * 4 + b
                jn = j + 2
                sn = (b + 2) % 4

                @pl.when(jn < npair)
                def _():
                    fire(jn, sn, True)

                drain(j, b)
            return carry

        lax.fori_loop(0, npair // 4, body, 0)
        for s in range(4):
            for _k in range(2):
                pltpu.make_async_copy(
                    rows.at[s, pl.ds(0, t0)],
                    out_hbm.at[0, 0, pl.ds(0, t0)],
                    sss[s],
                ).wait()

    return sc_k


def kernel(input_ids, embedding, species_embedding):
    B, S, T = input_ids.shape
    H = embedding.shape[1]
    t0 = (T // 8) * 8
    comb = _make_comb(embedding, species_embedding)
    ids_flat = input_ids.reshape(B * S * T)
    sc_k = _make_sc_kernel(B, S, T, H)
    big = sc_k(ids_flat, comb)
    tail = _tc_tail(input_ids[:, :, t0:], embedding, species_embedding)
    return lax.dynamic_update_slice(big, tail, (0, 0, t0, 0))
